# baseline (speedup 1.0000x reference)
import jax
import jax.numpy as jnp
from jax.experimental import pallas as pl

_N = 8192
_D = 1000
_K = 8
_INV_TEMP = 0.5
_BLOCK_ROWS = 1024


def _topk_mask_kernel(cls_ref, out_ref):
    x = cls_ref[:]
    m = jnp.max(x, axis=-1, keepdims=True)
    ew = jnp.exp((x - m) * _INV_TEMP)
    s = jnp.sum(ew, axis=-1, keepdims=True)
    ckey = (
        _D - jax.lax.broadcasted_iota(jnp.int32, ew.shape, 1)
    ).astype(jnp.float32)
    out = jnp.zeros_like(ew)
    for _ in range(_K):
        v = jnp.max(ew, axis=-1, keepdims=True)
        wk = jnp.max(jnp.where(ew >= v, ckey, 0.0), axis=-1, keepdims=True)
        onehot = ckey == wk
        out = jnp.where(onehot, v / s, out)
        s = s - v
        ew = jnp.where(onehot, 0.0, ew)
    out_ref[:] = out


def kernel(normu, cls):
    classes = pl.pallas_call(
        _topk_mask_kernel,
        grid=(_N // _BLOCK_ROWS,),
        in_specs=[pl.BlockSpec((_BLOCK_ROWS, _D), lambda i: (i, 0))],
        out_specs=pl.BlockSpec((_BLOCK_ROWS, _D), lambda i: (i, 0)),
        out_shape=jax.ShapeDtypeStruct((_N, _D), jnp.float32),
    )(cls)
    return (normu, classes)


# speedup vs baseline: 1.0241x; 1.0241x over previous
import jax
import jax.numpy as jnp
from jax.experimental import pallas as pl

_N = 8192
_D = 1000
_K = 8
_INV_TEMP = 0.5
_BLOCK_ROWS = 512


def _topk_mask_kernel(cls_ref, out_ref):
    x = cls_ref[:]
    m = jnp.max(x, axis=-1, keepdims=True)
    ew = jnp.exp((x - m) * _INV_TEMP)
    s = jnp.sum(ew, axis=-1, keepdims=True)
    ckey = (
        _D - jax.lax.broadcasted_iota(jnp.int32, ew.shape, 1)
    ).astype(jnp.float32)
    v = jnp.ones_like(s)
    coefs, wks = [], []
    for i in range(_K):
        wk = jnp.max(jnp.where(ew >= v, ckey, 0.0), axis=-1, keepdims=True)
        coefs.append(v / s)
        wks.append(wk)
        s = s - v
        if i < _K - 1:
            ew = jnp.where(ckey == wk, 0.0, ew)
            v = jnp.max(ew, axis=-1, keepdims=True)
    out = jnp.zeros_like(ew)
    for wk, c in zip(wks, coefs):
        out = jnp.where(ckey == wk, c, out)
    out_ref[:] = out


def kernel(normu, cls):
    classes = pl.pallas_call(
        _topk_mask_kernel,
        grid=(_N // _BLOCK_ROWS,),
        in_specs=[pl.BlockSpec((_BLOCK_ROWS, _D), lambda i: (i, 0))],
        out_specs=pl.BlockSpec((_BLOCK_ROWS, _D), lambda i: (i, 0)),
        out_shape=jax.ShapeDtypeStruct((_N, _D), jnp.float32),
    )(cls)
    return (normu, classes)


# speedup vs baseline: 1.0596x; 1.0346x over previous
import jax
import jax.numpy as jnp
from jax.experimental import pallas as pl

_N = 8192
_D = 1000
_K = 8
_INV_TEMP = 0.5
_BLOCK_ROWS = 512


def _topk_mask_kernel(cls_ref, out_ref):
    x = cls_ref[:]
    m = jnp.max(x, axis=-1, keepdims=True)
    ew = jnp.exp((x - m) * _INV_TEMP)
    s = jnp.sum(ew, axis=-1, keepdims=True)
    ckey = (
        _D - jax.lax.broadcasted_iota(jnp.int32, ew.shape, 1)
    ).astype(jnp.float32)
    v = jnp.ones_like(s)
    for i in range(_K):
        wk = jnp.max(jnp.where(ew >= v, ckey, 0.0), axis=-1, keepdims=True)
        ew = jnp.where(ckey == wk, -v / s, ew)
        s = s - v
        if i < _K - 1:
            v = jnp.max(ew, axis=-1, keepdims=True)
    out_ref[:] = -jnp.minimum(ew, 0.0)


def kernel(normu, cls):
    classes = pl.pallas_call(
        _topk_mask_kernel,
        grid=(_N // _BLOCK_ROWS,),
        in_specs=[pl.BlockSpec((_BLOCK_ROWS, _D), lambda i: (i, 0))],
        out_specs=pl.BlockSpec((_BLOCK_ROWS, _D), lambda i: (i, 0)),
        out_shape=jax.ShapeDtypeStruct((_N, _D), jnp.float32),
    )(cls)
    return (normu, classes)
